# baseline (speedup 1.0000x reference)
import functools

import jax
import jax.numpy as jnp
from jax import lax
from jax.experimental import pallas as pl
from jax.experimental.pallas import tpu as pltpu
from jax.experimental.pallas import tpu_sc as plsc

_INCREASE_DELTA = 0.05
_ABS_THRESHOLD = 0.8

_B = 16
_S = 4096
_L = 16
_NCHUNK = _S // _L
_PAD = _L
_NEG = -3e38
_POS = 3e38


def _seg_body(ent_hbm, seg_hbm, pem_hbm, fb_hbm, row_v, seg_v, pem_v, fb_v):
    wid = lax.axis_index("s")

    rb = wid * _S
    row_v[pl.ds(0, _L)] = jnp.full((_L,), _NEG, jnp.float32)
    pltpu.sync_copy(ent_hbm.at[pl.ds(rb, _S)], row_v.at[pl.ds(_PAD, _S)])
    row_v[pl.ds(_PAD + _S, _L)] = jnp.full((_L,), _POS, jnp.float32)

    lane = lax.iota(jnp.int32, _L)
    rlane = 15 - lane

    def chunk(i, carry):
        carry_sum, carry_max = carry
        base = _PAD + i * _L
        prev = row_v[pl.ds(base - 1, _L)]
        e = row_v[pl.ds(base, _L)]
        nxt = row_v[pl.ds(base + 1, _L)]
        inc = (e > prev + _INCREASE_DELTA) | (e > _ABS_THRESHOLD)
        pem = (nxt > e + _INCREASE_DELTA) | (nxt > _ABS_THRESHOLD)
        inc_i = inc.astype(jnp.int32)
        cs = plsc.cumsum(inc_i)
        seg = cs + (carry_sum - 1)
        pos = i * _L + lane
        fp = jnp.where(inc, pos, 0)
        cm = plsc.cummax(fp)
        fb = jnp.maximum(cm, carry_max)
        off = i * _L
        seg_v[pl.ds(off, _L)] = seg
        pem_v[pl.ds(off, _L)] = pem.astype(jnp.int32)
        fb_v[pl.ds(off, _L)] = fb
        cnt = plsc.all_reduce_population_count(inc)
        new_sum = carry_sum + cnt
        ffs = plsc.all_reduce_ffs(lax.rev(inc_i, (0,)) != 0)
        last_pos = (i * _L + 15) - ffs
        new_max = jnp.where(cnt > 0, last_pos, carry_max)
        return new_sum, new_max

    _UNROLL = 8

    def chunk_group(g, carry):
        for u in range(_UNROLL):
            carry = chunk(g * _UNROLL + u, carry)
        return carry

    lax.fori_loop(
        0, _NCHUNK // _UNROLL, chunk_group,
        (jnp.zeros((_L,), jnp.int32), jnp.zeros((_L,), jnp.int32)),
    )

    pltpu.sync_copy(seg_v, seg_hbm.at[pl.ds(rb, _S)])
    pltpu.sync_copy(pem_v, pem_hbm.at[pl.ds(rb, _S)])
    pltpu.sync_copy(fb_v, fb_hbm.at[pl.ds(rb, _S)])


@jax.jit
def _segmenter(entropy_bits):
    mesh = plsc.VectorSubcoreMesh(
        core_axis_name="c", subcore_axis_name="s", num_cores=1, num_subcores=16
    )
    out = jax.ShapeDtypeStruct((_B * _S,), jnp.int32)
    run = functools.partial(
        pl.kernel,
        out_type=(out, out, out),
        mesh=mesh,
        compiler_params=pltpu.CompilerParams(
            needs_layout_passes=False, skip_device_barrier=True
        ),
        scratch_types=[
            pltpu.VMEM((_PAD + _S + _L,), jnp.float32),
            pltpu.VMEM((_S,), jnp.int32),
            pltpu.VMEM((_S,), jnp.int32),
            pltpu.VMEM((_S,), jnp.int32),
        ],
    )(_seg_body)
    seg, pem, fb = run(entropy_bits.reshape(_B * _S))
    return (
        seg.reshape(_B, _S),
        pem.reshape(_B, _S) != 0,
        fb.reshape(_B, _S),
    )


def kernel(entropy_bits):
    return _segmenter(entropy_bits)


# speedup vs baseline: 1.1695x; 1.1695x over previous
import functools

import jax
import jax.numpy as jnp
from jax import lax
from jax.experimental import pallas as pl
from jax.experimental.pallas import tpu as pltpu
from jax.experimental.pallas import tpu_sc as plsc

_INCREASE_DELTA = 0.05
_ABS_THRESHOLD = 0.8

_B = 16
_S = 4096
_L = 16
_NCHUNK = _S // _L
_PAD = _L
_NEG = -3e38
_POS = 3e38


def _seg_body(ent_hbm, seg_hbm, pem_hbm, fb_hbm, row_v, seg_v, pem_v, fb_v):
    wid = lax.axis_index("s")

    rb = wid * _S
    row_v[pl.ds(0, _L)] = jnp.full((_L,), _NEG, jnp.float32)
    pltpu.sync_copy(ent_hbm.at[pl.ds(rb, _S)], row_v.at[pl.ds(_PAD, _S)])
    row_v[pl.ds(_PAD + _S, _L)] = jnp.full((_L,), _POS, jnp.float32)

    lane = lax.iota(jnp.int32, _L)
    rlane = 15 - lane

    @functools.partial(
        plsc.parallel_loop, 0, _NCHUNK, unroll=1,
        carry=(jnp.zeros((_L,), jnp.int32), jnp.zeros((_L,), jnp.int32)),
    )
    def chunk(i, carry):
        carry_sum, carry_max = carry
        base = _PAD + i * _L
        prev = row_v[pl.ds(base - 1, _L)]
        e = row_v[pl.ds(base, _L)]
        nxt = row_v[pl.ds(base + 1, _L)]
        inc = (e > prev + _INCREASE_DELTA) | (e > _ABS_THRESHOLD)
        pem = (nxt > e + _INCREASE_DELTA) | (nxt > _ABS_THRESHOLD)
        inc_i = inc.astype(jnp.int32)
        cs = plsc.cumsum(inc_i)
        seg = cs + (carry_sum - 1)
        pos = i * _L + lane
        fp = jnp.where(inc, pos, 0)
        cm = plsc.cummax(fp)
        fb = jnp.maximum(cm, carry_max)
        off = i * _L
        seg_v[pl.ds(off, _L)] = seg
        pem_v[pl.ds(off, _L)] = pem.astype(jnp.int32)
        fb_v[pl.ds(off, _L)] = fb
        cnt = plsc.all_reduce_population_count(inc)
        new_sum = carry_sum + cnt
        ffs = plsc.all_reduce_ffs(lax.rev(inc_i, (0,)) != 0)
        last_pos = (i * _L + 15) - ffs
        new_max = jnp.where(cnt > 0, last_pos, carry_max)
        return new_sum, new_max

    pltpu.sync_copy(seg_v, seg_hbm.at[pl.ds(rb, _S)])
    pltpu.sync_copy(pem_v, pem_hbm.at[pl.ds(rb, _S)])
    pltpu.sync_copy(fb_v, fb_hbm.at[pl.ds(rb, _S)])


@jax.jit
def _segmenter(entropy_bits):
    mesh = plsc.VectorSubcoreMesh(
        core_axis_name="c", subcore_axis_name="s", num_cores=1, num_subcores=16
    )
    out = jax.ShapeDtypeStruct((_B * _S,), jnp.int32)
    run = functools.partial(
        pl.kernel,
        out_type=(out, out, out),
        mesh=mesh,
        compiler_params=pltpu.CompilerParams(
            needs_layout_passes=False, skip_device_barrier=True
        ),
        scratch_types=[
            pltpu.VMEM((_PAD + _S + _L,), jnp.float32),
            pltpu.VMEM((_S,), jnp.int32),
            pltpu.VMEM((_S,), jnp.int32),
            pltpu.VMEM((_S,), jnp.int32),
        ],
    )(_seg_body)
    seg, pem, fb = run(entropy_bits.reshape(_B * _S))
    return (
        seg.reshape(_B, _S),
        pem.reshape(_B, _S) != 0,
        fb.reshape(_B, _S),
    )


def kernel(entropy_bits):
    return _segmenter(entropy_bits)
